# Initial kernel scaffold; baseline (speedup 1.0000x reference)
import jax
import jax.numpy as jnp
from jax.experimental import pallas as pl


def kernel(embeddings, edge_index, Wl1, bl1, Wr1, Wl2, bl2, Wr2):
    raise NotImplementedError("write your pallas kernel here")


# speedup vs baseline: 4.5689x; 4.5689x over previous
import jax
import jax.numpy as jnp
from jax import lax
from jax.experimental import pallas as pl
from jax.experimental.pallas import tpu as pltpu
from jax.experimental.pallas import tpu_sc as plsc

_N = 10000
_NP = 10240
_D = 128
_E = 320000
_NC = 2
_NS = 16
_NW = _NC * _NS
_EPW = _E // _NW
_CH = 80
_NCHUNK = _EPW // _CH
_RPS = _NP // _NS
_CW = 16


def _sc_aggregate(x, src, dst, z128, z16, ones):
    mesh = plsc.VectorSubcoreMesh(core_axis_name="c", subcore_axis_name="s",
                                  num_cores=_NC, num_subcores=_NS)

    def body(x_hbm, src_hbm, dst_hbm, z128_hbm, z16_hbm, ones_hbm,
             agg_hbm, cnt_hbm, src_v, dst_v, rows_v, ones_v, sml_v,
             sem, agg_sh, cnt_sh):
        cid = lax.axis_index("c")
        sid = lax.axis_index("s")
        r0 = sid * _RPS

        pltpu.sync_copy(z128_hbm, rows_v)
        pltpu.sync_copy(z16_hbm, sml_v)

        def zstep(j, carry):
            pltpu.sync_copy(rows_v, agg_sh.at[pl.ds(r0 + j * _CH, _CH)])
            pltpu.sync_copy(sml_v, cnt_sh.at[pl.ds(r0 + j * _CH, _CH)])
            return carry

        lax.fori_loop(0, _RPS // _CH, zstep, 0)
        pltpu.sync_copy(ones_hbm, ones_v)
        plsc.subcore_barrier()

        ebase = (sid * _NC + cid) * _EPW

        def step(i, carry):
            e0 = ebase + i * _CH
            pltpu.sync_copy(src_hbm.at[pl.ds(e0, _CH)], src_v)
            pltpu.sync_copy(dst_hbm.at[pl.ds(e0, _CH)], dst_v)
            pltpu.async_copy(x_hbm.at[src_v], rows_v, sem).wait()
            pltpu.sync_copy(rows_v, agg_sh.at[dst_v], add=True)
            pltpu.sync_copy(ones_v, cnt_sh.at[dst_v], add=True)
            return carry

        lax.fori_loop(0, _NCHUNK, step, 0)

        plsc.subcore_barrier()

        def wstep(j, carry):
            w0 = r0 + j * _CH
            pltpu.sync_copy(agg_sh.at[pl.ds(w0, _CH)], rows_v)
            pltpu.sync_copy(rows_v, agg_hbm.at[cid, pl.ds(w0, _CH)])
            pltpu.sync_copy(cnt_sh.at[pl.ds(w0, _CH)], sml_v)
            pltpu.sync_copy(sml_v, cnt_hbm.at[cid, pl.ds(w0, _CH)])
            return carry

        lax.fori_loop(0, _RPS // _CH, wstep, 0)

    f = pl.kernel(
        body,
        out_type=(jax.ShapeDtypeStruct((_NC, _NP, _D), jnp.float32),
                  jax.ShapeDtypeStruct((_NC, _NP, _CW), jnp.float32)),
        mesh=mesh,
        compiler_params=pltpu.CompilerParams(use_tc_tiling_on_sc=False),
        scratch_types=(
            pltpu.VMEM((_CH,), jnp.int32),
            pltpu.VMEM((_CH,), jnp.int32),
            pltpu.VMEM((_CH, _D), jnp.float32),
            pltpu.VMEM((_CH, _CW), jnp.float32),
            pltpu.VMEM((_CH, _CW), jnp.float32),
            pltpu.SemaphoreType.DMA,
            pltpu.VMEM_SHARED((_NP, _D), jnp.float32),
            pltpu.VMEM_SHARED((_NP, _CW), jnp.float32),
        ),
    )
    return f(x, src, dst, z128, z16, ones)


def _tc_dense(agg0, agg1, cnt0, cnt1, x, wl, bl, wr):
    bn = 1024

    def body(a0, a1, c0, c1, x_r, wl_r, bl_r, wr_r, o_r):
        agg = a0[...] + a1[...]
        cnt = (c0[...] + c1[...])[:, :1]
        mean = agg / jnp.maximum(cnt, 1.0)
        h = (lax.dot_general(mean, wl_r[...], (((1,), (1,)), ((), ())),
                             preferred_element_type=jnp.float32,
                             precision=lax.Precision.HIGHEST)
             + bl_r[...]
             + lax.dot_general(x_r[...], wr_r[...], (((1,), (1,)), ((), ())),
                               preferred_element_type=jnp.float32,
                               precision=lax.Precision.HIGHEST))
        nrm = jnp.sqrt(jnp.sum(h * h, axis=1, keepdims=True))
        o_r[...] = h / jnp.maximum(nrm, 1e-12)

    return pl.pallas_call(
        body,
        grid=(_NP // bn,),
        in_specs=[
            pl.BlockSpec((bn, _D), lambda i: (i, 0)),
            pl.BlockSpec((bn, _D), lambda i: (i, 0)),
            pl.BlockSpec((bn, _CW), lambda i: (i, 0)),
            pl.BlockSpec((bn, _CW), lambda i: (i, 0)),
            pl.BlockSpec((bn, _D), lambda i: (i, 0)),
            pl.BlockSpec((_D, _D), lambda i: (0, 0)),
            pl.BlockSpec((1, _D), lambda i: (0, 0)),
            pl.BlockSpec((_D, _D), lambda i: (0, 0)),
        ],
        out_specs=pl.BlockSpec((bn, _D), lambda i: (i, 0)),
        out_shape=jax.ShapeDtypeStruct((_NP, _D), jnp.float32),
    )(agg0, agg1, cnt0, cnt1, x, wl, bl, wr)


def kernel(embeddings, edge_index, Wl1, bl1, Wr1, Wl2, bl2, Wr2):
    src = edge_index[0]
    dst = edge_index[1]
    z128 = jnp.zeros((_CH, _D), jnp.float32)
    z16 = jnp.zeros((_CH, _CW), jnp.float32)
    ones = jnp.ones((_CH, _CW), jnp.float32)
    xp = jnp.concatenate(
        [embeddings, jnp.zeros((_NP - _N, _D), jnp.float32)], axis=0)

    agg, cnt = _sc_aggregate(xp, src, dst, z128, z16, ones)
    h1 = _tc_dense(agg[0], agg[1], cnt[0], cnt[1], xp,
                   Wl1, jnp.reshape(bl1, (1, _D)), Wr1)
    agg2, cnt2 = _sc_aggregate(h1, src, dst, z128, z16, ones)
    h2 = _tc_dense(agg2[0], agg2[1], cnt2[0], cnt2[1], h1,
                   Wl2, jnp.reshape(bl2, (1, _D)), Wr2)
    return h2[:_N]


# speedup vs baseline: 8.1127x; 1.7756x over previous
import jax
import jax.numpy as jnp
from jax import lax
from jax.experimental import pallas as pl
from jax.experimental.pallas import tpu as pltpu
from jax.experimental.pallas import tpu_sc as plsc

_N = 10000
_NP = 10240
_D = 128
_DA = 144
_E = 320000
_NC = 2
_NS = 16
_NW = _NC * _NS
_EPW = _E // _NW
_CH = 80
_NCHUNK = _EPW // _CH
_RPS = _NP // _NS


def _sc_aggregate(width, x, edges3, zrow):
    mesh = plsc.VectorSubcoreMesh(core_axis_name="c", subcore_axis_name="s",
                                  num_cores=_NC, num_subcores=_NS)

    def body(x_hbm, e_hbm, z_hbm, agg_hbm, idx_v, rows_v, sem, agg_sh):
        cid = lax.axis_index("c")
        sid = lax.axis_index("s")
        r0 = sid * _RPS

        pltpu.sync_copy(z_hbm, rows_v.at[0])

        def zstep(j, carry):
            pltpu.sync_copy(rows_v.at[0], agg_sh.at[pl.ds(r0 + j * _CH, _CH)])
            return carry

        lax.fori_loop(0, _RPS // _CH, zstep, 0)
        plsc.subcore_barrier()

        cbase = (sid * _NC + cid) * _NCHUNK

        def start_gather(c, p):
            pltpu.sync_copy(e_hbm.at[cbase + c], idx_v.at[p])
            pltpu.async_copy(x_hbm.at[idx_v.at[p, 0]], rows_v.at[p],
                             sem.at[p])

        start_gather(0, 0)
        start_gather(1, 1)

        def step(i, carry):
            p = lax.rem(i, 2)
            pltpu.make_async_copy(x_hbm.at[idx_v.at[p, 0]], rows_v.at[p],
                                  sem.at[p]).wait()
            pltpu.sync_copy(rows_v.at[p], agg_sh.at[idx_v.at[p, 1]], add=True)

            @pl.when(i + 2 < _NCHUNK)
            def _():
                start_gather(i + 2, p)

            return carry

        lax.fori_loop(0, _NCHUNK, step, 0)

        plsc.subcore_barrier()

        def wstep(j, carry):
            w0 = r0 + j * _CH
            pltpu.sync_copy(agg_sh.at[pl.ds(w0, _CH)], rows_v.at[0])
            pltpu.sync_copy(rows_v.at[0], agg_hbm.at[cid, pl.ds(w0, _CH)])
            return carry

        lax.fori_loop(0, _RPS // _CH, wstep, 0)

    f = pl.kernel(
        body,
        out_type=jax.ShapeDtypeStruct((_NC, _NP, width), jnp.float32),
        mesh=mesh,
        compiler_params=pltpu.CompilerParams(use_tc_tiling_on_sc=False),
        scratch_types=(
            pltpu.VMEM((2, 2, _CH), jnp.int32),
            pltpu.VMEM((2, _CH, width), jnp.float32),
            pltpu.SemaphoreType.DMA((2,)),
            pltpu.VMEM_SHARED((_NP, width), jnp.float32),
        ),
    )
    return f(x, edges3, zrow)


def _tc_dense(aggp, cntp, x, wl, bl, wr):
    bn = 1024
    wa = aggp.shape[-1]

    def body(a_r, c_r, x_r, wl_r, bl_r, wr_r, o_r):
        agg = a_r[0, :, :_D] + a_r[1, :, :_D]
        cnt = c_r[0, :, _D:_D + 1] + c_r[1, :, _D:_D + 1]
        mean = agg / jnp.maximum(cnt, 1.0)
        h = (lax.dot_general(mean, wl_r[...], (((1,), (1,)), ((), ())),
                             preferred_element_type=jnp.float32,
                             precision=lax.Precision.HIGHEST)
             + bl_r[...]
             + lax.dot_general(x_r[:, :_D], wr_r[...], (((1,), (1,)), ((), ())),
                               preferred_element_type=jnp.float32,
                               precision=lax.Precision.HIGHEST))
        nrm = jnp.sqrt(jnp.sum(h * h, axis=1, keepdims=True))
        o_r[...] = h / jnp.maximum(nrm, 1e-12)

    return pl.pallas_call(
        body,
        grid=(_NP // bn,),
        in_specs=[
            pl.BlockSpec((2, bn, wa), lambda i: (0, i, 0)),
            pl.BlockSpec((2, bn, _DA), lambda i: (0, i, 0)),
            pl.BlockSpec((bn, x.shape[-1]), lambda i: (i, 0)),
            pl.BlockSpec((_D, _D), lambda i: (0, 0)),
            pl.BlockSpec((1, _D), lambda i: (0, 0)),
            pl.BlockSpec((_D, _D), lambda i: (0, 0)),
        ],
        out_specs=pl.BlockSpec((bn, _D), lambda i: (i, 0)),
        out_shape=jax.ShapeDtypeStruct((_NP, _D), jnp.float32),
    )(aggp, cntp, x, wl, bl, wr)


def kernel(embeddings, edge_index, Wl1, bl1, Wr1, Wl2, bl2, Wr2):
    edges3 = jnp.stack([edge_index[0].reshape(_E // _CH, _CH),
                        edge_index[1].reshape(_E // _CH, _CH)], axis=1)
    xaug = jnp.concatenate(
        [embeddings,
         jnp.ones((_N, _DA - _D), jnp.float32)], axis=1)
    xaug = jnp.concatenate(
        [xaug, jnp.zeros((_NP - _N, _DA), jnp.float32)], axis=0)
    z144 = jnp.zeros((_CH, _DA), jnp.float32)
    z128 = jnp.zeros((_CH, _D), jnp.float32)

    agg1 = _sc_aggregate(_DA, xaug, edges3, z144)
    h1 = _tc_dense(agg1, agg1, xaug,
                   Wl1, jnp.reshape(bl1, (1, _D)), Wr1)
    agg2 = _sc_aggregate(_D, h1, edges3, z128)
    h2 = _tc_dense(agg2, agg1, h1,
                   Wl2, jnp.reshape(bl2, (1, _D)), Wr2)
    return h2[:_N]


# speedup vs baseline: 9.4627x; 1.1664x over previous
import jax
import jax.numpy as jnp
from jax import lax
from jax.experimental import pallas as pl
from jax.experimental.pallas import tpu as pltpu
from jax.experimental.pallas import tpu_sc as plsc

_N = 10000
_NP = 10240
_D = 128
_DA = 136
_E = 320000
_NC = 2
_NS = 16
_NW = _NC * _NS
_EPW = _E // _NW
_CH = 100
_NCHUNK = _EPW // _CH
_NIB = 4
_WB = 80
_RPS = _NP // _NS


def _sc_aggregate(width, x, edges3, zrow):
    mesh = plsc.VectorSubcoreMesh(core_axis_name="c", subcore_axis_name="s",
                                  num_cores=_NC, num_subcores=_NS)

    def body(x_hbm, e_hbm, z_hbm, agg_hbm, idx_v, rows_v, gsem, isem, agg_sh):
        cid = lax.axis_index("c")
        sid = lax.axis_index("s")
        r0 = sid * _RPS

        pltpu.sync_copy(z_hbm, rows_v.at[0, pl.ds(0, _WB)])

        def zstep(j, carry):
            pltpu.sync_copy(rows_v.at[0, pl.ds(0, _WB)],
                            agg_sh.at[pl.ds(r0 + j * _WB, _WB)])
            return carry

        lax.fori_loop(0, _RPS // _WB, zstep, 0)
        plsc.subcore_barrier()

        cbase = (sid * _NC + cid) * _NCHUNK

        def start_idx(c, q):
            pltpu.async_copy(e_hbm.at[cbase + c], idx_v.at[q], isem.at[q])

        def wait_idx(c, q):
            pltpu.make_async_copy(e_hbm.at[cbase + c], idx_v.at[q],
                                  isem.at[q]).wait()

        def start_gather(q, p):
            pltpu.async_copy(x_hbm.at[idx_v.at[q, 0]], rows_v.at[p],
                             gsem.at[p])

        def wait_gather(q, p):
            pltpu.make_async_copy(x_hbm.at[idx_v.at[q, 0]], rows_v.at[p],
                                  gsem.at[p]).wait()

        for c in range(3):
            start_idx(c, c)
        for c in range(2):
            wait_idx(c, c)
            start_gather(c, c)

        def step(i, carry):
            p = lax.rem(i, 2)
            q = lax.rem(i, _NIB)
            wait_gather(q, p)
            pltpu.sync_copy(rows_v.at[p], agg_sh.at[idx_v.at[q, 1]], add=True)

            @pl.when(i + 3 < _NCHUNK)
            def _():
                start_idx(i + 3, lax.rem(i + 3, _NIB))

            @pl.when(i + 2 < _NCHUNK)
            def _():
                qq = lax.rem(i + 2, _NIB)
                wait_idx(i + 2, qq)
                start_gather(qq, p)

            return carry

        lax.fori_loop(0, _NCHUNK, step, 0)

        plsc.subcore_barrier()

        def wstep(j, carry):
            w0 = r0 + j * _WB
            pltpu.sync_copy(agg_sh.at[pl.ds(w0, _WB)],
                            rows_v.at[0, pl.ds(0, _WB)])
            pltpu.sync_copy(rows_v.at[0, pl.ds(0, _WB)],
                            agg_hbm.at[cid, pl.ds(w0, _WB)])
            return carry

        lax.fori_loop(0, _RPS // _WB, wstep, 0)

    f = pl.kernel(
        body,
        out_type=jax.ShapeDtypeStruct((_NC, _NP, width), jnp.float32),
        mesh=mesh,
        compiler_params=pltpu.CompilerParams(use_tc_tiling_on_sc=False),
        scratch_types=(
            pltpu.VMEM((_NIB, 2, _CH), jnp.int32),
            pltpu.VMEM((2, _CH, width), jnp.float32),
            pltpu.SemaphoreType.DMA((2,)),
            pltpu.SemaphoreType.DMA((_NIB,)),
            pltpu.VMEM_SHARED((_NP, width), jnp.float32),
        ),
    )
    return f(x, edges3, zrow)


def _tc_dense(aggp, cntp, x, wl, bl, wr):
    bn = 1024
    wa = aggp.shape[-1]

    def body(a_r, c_r, x_r, wl_r, bl_r, wr_r, o_r):
        agg = a_r[0, :, :_D] + a_r[1, :, :_D]
        cnt = c_r[0, :, _D:_D + 1] + c_r[1, :, _D:_D + 1]
        mean = agg / jnp.maximum(cnt, 1.0)
        h = (lax.dot_general(mean, wl_r[...], (((1,), (1,)), ((), ())),
                             preferred_element_type=jnp.float32,
                             precision=lax.Precision.HIGHEST)
             + bl_r[...]
             + lax.dot_general(x_r[:, :_D], wr_r[...], (((1,), (1,)), ((), ())),
                               preferred_element_type=jnp.float32,
                               precision=lax.Precision.HIGHEST))
        nrm = jnp.sqrt(jnp.sum(h * h, axis=1, keepdims=True))
        o_r[...] = h / jnp.maximum(nrm, 1e-12)

    return pl.pallas_call(
        body,
        grid=(_NP // bn,),
        in_specs=[
            pl.BlockSpec((2, bn, wa), lambda i: (0, i, 0)),
            pl.BlockSpec((2, bn, _DA), lambda i: (0, i, 0)),
            pl.BlockSpec((bn, x.shape[-1]), lambda i: (i, 0)),
            pl.BlockSpec((_D, _D), lambda i: (0, 0)),
            pl.BlockSpec((1, _D), lambda i: (0, 0)),
            pl.BlockSpec((_D, _D), lambda i: (0, 0)),
        ],
        out_specs=pl.BlockSpec((bn, _D), lambda i: (i, 0)),
        out_shape=jax.ShapeDtypeStruct((_NP, _D), jnp.float32),
    )(aggp, cntp, x, wl, bl, wr)


def kernel(embeddings, edge_index, Wl1, bl1, Wr1, Wl2, bl2, Wr2):
    edges3 = jnp.stack([edge_index[0].reshape(_E // _CH, _CH),
                        edge_index[1].reshape(_E // _CH, _CH)], axis=1)
    xaug = jnp.concatenate(
        [embeddings,
         jnp.ones((_N, _DA - _D), jnp.float32)], axis=1)
    xaug = jnp.concatenate(
        [xaug, jnp.zeros((_NP - _N, _DA), jnp.float32)], axis=0)
    z144 = jnp.zeros((_WB, _DA), jnp.float32)
    z128 = jnp.zeros((_WB, _D), jnp.float32)

    agg1 = _sc_aggregate(_DA, xaug, edges3, z144)
    h1 = _tc_dense(agg1, agg1, xaug,
                   Wl1, jnp.reshape(bl1, (1, _D)), Wr1)
    agg2 = _sc_aggregate(_D, h1, edges3, z128)
    h2 = _tc_dense(agg2, agg1, h1,
                   Wl2, jnp.reshape(bl2, (1, _D)), Wr2)
    return h2[:_N]


# speedup vs baseline: 10.2043x; 1.0784x over previous
import jax
import jax.numpy as jnp
from jax import lax
from jax.experimental import pallas as pl
from jax.experimental.pallas import tpu as pltpu
from jax.experimental.pallas import tpu_sc as plsc

_N = 10000
_NP = 10240
_D = 128
_DA = 136
_E = 320000
_NC = 2
_NS = 16
_NW = _NC * _NS
_EPW = _E // _NW
_CH = 100
_NCHUNK = _EPW // _CH
_NIB = 4
_WB = 80
_RPS = _NP // _NS


def _sc_aggregate(width, x, edges3, zrow):
    mesh = plsc.VectorSubcoreMesh(core_axis_name="c", subcore_axis_name="s",
                                  num_cores=_NC, num_subcores=_NS)

    def body(x_hbm, e_hbm, z_hbm, agg_hbm, idx_v, rows_v, gsem, isem, ssem,
             agg_sh):
        cid = lax.axis_index("c")
        sid = lax.axis_index("s")
        r0 = sid * _RPS

        pltpu.sync_copy(z_hbm, rows_v.at[0, pl.ds(0, _WB)])

        def zstep(j, carry):
            pltpu.sync_copy(rows_v.at[0, pl.ds(0, _WB)],
                            agg_sh.at[pl.ds(r0 + j * _WB, _WB)])
            return carry

        lax.fori_loop(0, _RPS // _WB, zstep, 0)
        plsc.subcore_barrier()

        cbase = (sid * _NC + cid) * _NCHUNK

        def start_idx(c, q):
            pltpu.async_copy(e_hbm.at[cbase + c], idx_v.at[q], isem.at[q])

        def wait_idx(c, q):
            pltpu.make_async_copy(e_hbm.at[cbase + c], idx_v.at[q],
                                  isem.at[q]).wait()

        def start_gather(q, p):
            pltpu.async_copy(x_hbm.at[idx_v.at[q, 0]], rows_v.at[p],
                             gsem.at[p])

        def wait_gather(q, p):
            pltpu.make_async_copy(x_hbm.at[idx_v.at[q, 0]], rows_v.at[p],
                                  gsem.at[p]).wait()

        def start_scatter(q, p):
            pltpu.async_copy(rows_v.at[p], agg_sh.at[idx_v.at[q, 1]],
                             ssem.at[p], add=True)

        def wait_scatter(q, p):
            pltpu.make_async_copy(rows_v.at[p], agg_sh.at[idx_v.at[q, 1]],
                                  ssem.at[p]).wait()

        for c in range(3):
            start_idx(c, c)
        for c in range(2):
            wait_idx(c, c)
            start_gather(c, c)

        def step(i, carry):
            p = lax.rem(i, 3)
            q = lax.rem(i, _NIB)
            wait_gather(q, p)
            start_scatter(q, p)

            @pl.when(i >= 1)
            def _():
                wait_scatter(lax.rem(i - 1, _NIB), lax.rem(i + 2, 3))

            @pl.when(i + 3 < _NCHUNK)
            def _():
                start_idx(i + 3, lax.rem(i + 3, _NIB))

            @pl.when(i + 2 < _NCHUNK)
            def _():
                qq = lax.rem(i + 2, _NIB)
                wait_idx(i + 2, qq)
                start_gather(qq, lax.rem(i + 2, 3))

            return carry

        lax.fori_loop(0, _NCHUNK, step, 0)

        wait_scatter((_NCHUNK - 1) % _NIB, (_NCHUNK - 1) % 3)

        plsc.subcore_barrier()

        def wstep(j, carry):
            w0 = r0 + j * _WB
            pltpu.sync_copy(agg_sh.at[pl.ds(w0, _WB)],
                            rows_v.at[0, pl.ds(0, _WB)])
            pltpu.sync_copy(rows_v.at[0, pl.ds(0, _WB)],
                            agg_hbm.at[cid, pl.ds(w0, _WB)])
            return carry

        lax.fori_loop(0, _RPS // _WB, wstep, 0)

    f = pl.kernel(
        body,
        out_type=jax.ShapeDtypeStruct((_NC, _NP, width), jnp.float32),
        mesh=mesh,
        compiler_params=pltpu.CompilerParams(use_tc_tiling_on_sc=False),
        scratch_types=(
            pltpu.VMEM((_NIB, 2, _CH), jnp.int32),
            pltpu.VMEM((3, _CH, width), jnp.float32),
            pltpu.SemaphoreType.DMA((3,)),
            pltpu.SemaphoreType.DMA((_NIB,)),
            pltpu.SemaphoreType.DMA((3,)),
            pltpu.VMEM_SHARED((_NP, width), jnp.float32),
        ),
    )
    return f(x, edges3, zrow)


def _tc_dense(aggp, cntp, x, wl, bl, wr):
    bn = 1024
    wa = aggp.shape[-1]

    def body(a_r, c_r, x_r, wl_r, bl_r, wr_r, o_r):
        agg = a_r[0, :, :_D] + a_r[1, :, :_D]
        cnt = c_r[0, :, _D:_D + 1] + c_r[1, :, _D:_D + 1]
        mean = agg / jnp.maximum(cnt, 1.0)
        h = (lax.dot_general(mean, wl_r[...], (((1,), (1,)), ((), ())),
                             preferred_element_type=jnp.float32,
                             precision=lax.Precision.HIGHEST)
             + bl_r[...]
             + lax.dot_general(x_r[:, :_D], wr_r[...], (((1,), (1,)), ((), ())),
                               preferred_element_type=jnp.float32,
                               precision=lax.Precision.HIGHEST))
        nrm = jnp.sqrt(jnp.sum(h * h, axis=1, keepdims=True))
        o_r[...] = h / jnp.maximum(nrm, 1e-12)

    return pl.pallas_call(
        body,
        grid=(_NP // bn,),
        in_specs=[
            pl.BlockSpec((2, bn, wa), lambda i: (0, i, 0)),
            pl.BlockSpec((2, bn, _DA), lambda i: (0, i, 0)),
            pl.BlockSpec((bn, x.shape[-1]), lambda i: (i, 0)),
            pl.BlockSpec((_D, _D), lambda i: (0, 0)),
            pl.BlockSpec((1, _D), lambda i: (0, 0)),
            pl.BlockSpec((_D, _D), lambda i: (0, 0)),
        ],
        out_specs=pl.BlockSpec((bn, _D), lambda i: (i, 0)),
        out_shape=jax.ShapeDtypeStruct((_NP, _D), jnp.float32),
    )(aggp, cntp, x, wl, bl, wr)


def kernel(embeddings, edge_index, Wl1, bl1, Wr1, Wl2, bl2, Wr2):
    edges3 = jnp.stack([edge_index[0].reshape(_E // _CH, _CH),
                        edge_index[1].reshape(_E // _CH, _CH)], axis=1)
    xaug = jnp.concatenate(
        [embeddings,
         jnp.ones((_N, _DA - _D), jnp.float32)], axis=1)
    xaug = jnp.concatenate(
        [xaug, jnp.zeros((_NP - _N, _DA), jnp.float32)], axis=0)
    z144 = jnp.zeros((_WB, _DA), jnp.float32)
    z128 = jnp.zeros((_WB, _D), jnp.float32)

    agg1 = _sc_aggregate(_DA, xaug, edges3, z144)
    h1 = _tc_dense(agg1, agg1, xaug,
                   Wl1, jnp.reshape(bl1, (1, _D)), Wr1)
    agg2 = _sc_aggregate(_D, h1, edges3, z128)
    h2 = _tc_dense(agg2, agg1, h1,
                   Wl2, jnp.reshape(bl2, (1, _D)), Wr2)
    return h2[:_N]


# speedup vs baseline: 10.4271x; 1.0218x over previous
import jax
import jax.numpy as jnp
from jax import lax
from jax.experimental import pallas as pl
from jax.experimental.pallas import tpu as pltpu
from jax.experimental.pallas import tpu_sc as plsc

_N = 10000
_NP = 10240
_D = 128
_DA = 136
_E = 320000
_NC = 2
_NS = 16
_NW = _NC * _NS
_EPW = _E // _NW
_CH = 100
_NCHUNK = _EPW // _CH
_NIB = 4
_WB = 80
_RPS = _NP // _NS


def _sc_aggregate(width, x, edges3, zrow):
    mesh = plsc.VectorSubcoreMesh(core_axis_name="c", subcore_axis_name="s",
                                  num_cores=_NC, num_subcores=_NS)

    def body(x_hbm, e_hbm, z_hbm, agg_hbm, idx_v, rows_v, gsem, isem, ssem,
             agg_sh):
        cid = lax.axis_index("c")
        sid = lax.axis_index("s")
        r0 = sid * _RPS

        pltpu.sync_copy(z_hbm, rows_v.at[0, pl.ds(0, _WB)])

        def zstep(j, carry):
            pltpu.async_copy(rows_v.at[0, pl.ds(0, _WB)],
                             agg_sh.at[pl.ds(r0 + j * _WB, _WB)], gsem.at[0])
            return carry

        lax.fori_loop(0, _RPS // _WB, zstep, 0)

        def zdrain(j, carry):
            pltpu.make_async_copy(rows_v.at[0, pl.ds(0, _WB)],
                                  agg_sh.at[pl.ds(r0, _WB)], gsem.at[0]).wait()
            return carry

        lax.fori_loop(0, _RPS // _WB, zdrain, 0)
        plsc.subcore_barrier()

        cbase = (sid * _NC + cid) * _NCHUNK

        def start_idx(c, q):
            pltpu.async_copy(e_hbm.at[cbase + c], idx_v.at[q], isem.at[q])

        def wait_idx(c, q):
            pltpu.make_async_copy(e_hbm.at[cbase + c], idx_v.at[q],
                                  isem.at[q]).wait()

        def start_gather(q, p):
            pltpu.async_copy(x_hbm.at[idx_v.at[q, 0]], rows_v.at[p],
                             gsem.at[p])

        def wait_gather(q, p):
            pltpu.make_async_copy(x_hbm.at[idx_v.at[q, 0]], rows_v.at[p],
                                  gsem.at[p]).wait()

        def start_scatter(q, p):
            pltpu.async_copy(rows_v.at[p], agg_sh.at[idx_v.at[q, 1]],
                             ssem.at[p], add=True)

        def wait_scatter(q, p):
            pltpu.make_async_copy(rows_v.at[p], agg_sh.at[idx_v.at[q, 1]],
                                  ssem.at[p]).wait()

        for c in range(3):
            start_idx(c, c)
        for c in range(2):
            wait_idx(c, c)
            start_gather(c, c)

        def step(i, carry):
            p = lax.rem(i, 3)
            q = lax.rem(i, _NIB)
            wait_gather(q, p)
            start_scatter(q, p)

            @pl.when(i >= 1)
            def _():
                wait_scatter(lax.rem(i - 1, _NIB), lax.rem(i + 2, 3))

            @pl.when(i + 3 < _NCHUNK)
            def _():
                start_idx(i + 3, lax.rem(i + 3, _NIB))

            @pl.when(i + 2 < _NCHUNK)
            def _():
                qq = lax.rem(i + 2, _NIB)
                wait_idx(i + 2, qq)
                start_gather(qq, lax.rem(i + 2, 3))

            return carry

        lax.fori_loop(0, _NCHUNK, step, 0)

        wait_scatter((_NCHUNK - 1) % _NIB, (_NCHUNK - 1) % 3)

        plsc.subcore_barrier()

        def rd(j, p):
            pltpu.async_copy(agg_sh.at[pl.ds(r0 + j * _WB, _WB)],
                             rows_v.at[p, pl.ds(0, _WB)], gsem.at[p])

        def rd_wait(j, p):
            pltpu.make_async_copy(agg_sh.at[pl.ds(r0 + j * _WB, _WB)],
                                  rows_v.at[p, pl.ds(0, _WB)],
                                  gsem.at[p]).wait()

        def wr(j, p):
            pltpu.async_copy(rows_v.at[p, pl.ds(0, _WB)],
                             agg_hbm.at[cid, pl.ds(r0 + j * _WB, _WB)],
                             ssem.at[p])

        def wr_wait(j, p):
            pltpu.make_async_copy(rows_v.at[p, pl.ds(0, _WB)],
                                  agg_hbm.at[cid, pl.ds(r0 + j * _WB, _WB)],
                                  ssem.at[p]).wait()

        rd(0, 0)
        nw = _RPS // _WB

        def wstep(j, carry):
            p = lax.rem(j, 2)
            rd_wait(j, p)
            wr(j, p)

            @pl.when(j + 1 < nw)
            def _():
                p1 = lax.rem(j + 1, 2)

                @pl.when(j >= 1)
                def _():
                    wr_wait(j - 1, p1)

                rd(j + 1, p1)

            return carry

        lax.fori_loop(0, nw, wstep, 0)
        wr_wait(nw - 2, (nw - 2) % 2)
        wr_wait(nw - 1, (nw - 1) % 2)

    f = pl.kernel(
        body,
        out_type=jax.ShapeDtypeStruct((_NC, _NP, width), jnp.float32),
        mesh=mesh,
        compiler_params=pltpu.CompilerParams(use_tc_tiling_on_sc=False),
        scratch_types=(
            pltpu.VMEM((_NIB, 2, _CH), jnp.int32),
            pltpu.VMEM((3, _CH, width), jnp.float32),
            pltpu.SemaphoreType.DMA((3,)),
            pltpu.SemaphoreType.DMA((_NIB,)),
            pltpu.SemaphoreType.DMA((3,)),
            pltpu.VMEM_SHARED((_NP, width), jnp.float32),
        ),
    )
    return f(x, edges3, zrow)


def _tc_dense1(aggp, x, wl, bl, wr):
    bn = 1000

    def body(a_r, x_r, wl_r, bl_r, wr_r, o_r, c_r):
        agg = a_r[0, :, :_D] + a_r[1, :, :_D]
        cnt = a_r[0, :, _D:_D + 1] + a_r[1, :, _D:_D + 1]
        mean = agg / jnp.maximum(cnt, 1.0)
        h = (lax.dot_general(mean, wl_r[...], (((1,), (1,)), ((), ())),
                             preferred_element_type=jnp.float32,
                             precision=lax.Precision.HIGHEST)
             + bl_r[...]
             + lax.dot_general(x_r[:, :_D], wr_r[...], (((1,), (1,)), ((), ())),
                               preferred_element_type=jnp.float32,
                               precision=lax.Precision.HIGHEST))
        nrm = jnp.sqrt(jnp.sum(h * h, axis=1, keepdims=True))
        o_r[...] = h / jnp.maximum(nrm, 1e-12)
        c_r[...] = jnp.broadcast_to(cnt, (bn, 8))

    return pl.pallas_call(
        body,
        grid=(_N // bn,),
        in_specs=[
            pl.BlockSpec((2, bn, _DA), lambda i: (0, i, 0)),
            pl.BlockSpec((bn, _DA), lambda i: (i, 0)),
            pl.BlockSpec((_D, _D), lambda i: (0, 0)),
            pl.BlockSpec((1, _D), lambda i: (0, 0)),
            pl.BlockSpec((_D, _D), lambda i: (0, 0)),
        ],
        out_specs=[pl.BlockSpec((bn, _D), lambda i: (i, 0)),
                   pl.BlockSpec((bn, 8), lambda i: (i, 0))],
        out_shape=[jax.ShapeDtypeStruct((_N, _D), jnp.float32),
                   jax.ShapeDtypeStruct((_N, 8), jnp.float32)],
    )(aggp, x, wl, bl, wr)


def _tc_dense2(aggp, cnt8, x, wl, bl, wr):
    bn = 1000

    def body(a_r, c_r, x_r, wl_r, bl_r, wr_r, o_r):
        agg = a_r[0] + a_r[1]
        cnt = c_r[:, :1]
        mean = agg / jnp.maximum(cnt, 1.0)
        h = (lax.dot_general(mean, wl_r[...], (((1,), (1,)), ((), ())),
                             preferred_element_type=jnp.float32,
                             precision=lax.Precision.HIGHEST)
             + bl_r[...]
             + lax.dot_general(x_r[...], wr_r[...], (((1,), (1,)), ((), ())),
                               preferred_element_type=jnp.float32,
                               precision=lax.Precision.HIGHEST))
        nrm = jnp.sqrt(jnp.sum(h * h, axis=1, keepdims=True))
        o_r[...] = h / jnp.maximum(nrm, 1e-12)

    return pl.pallas_call(
        body,
        grid=(_N // bn,),
        in_specs=[
            pl.BlockSpec((2, bn, _D), lambda i: (0, i, 0)),
            pl.BlockSpec((bn, 8), lambda i: (i, 0)),
            pl.BlockSpec((bn, _D), lambda i: (i, 0)),
            pl.BlockSpec((_D, _D), lambda i: (0, 0)),
            pl.BlockSpec((1, _D), lambda i: (0, 0)),
            pl.BlockSpec((_D, _D), lambda i: (0, 0)),
        ],
        out_specs=pl.BlockSpec((bn, _D), lambda i: (i, 0)),
        out_shape=jax.ShapeDtypeStruct((_N, _D), jnp.float32),
    )(aggp, cnt8, x, wl, bl, wr)


def kernel(embeddings, edge_index, Wl1, bl1, Wr1, Wl2, bl2, Wr2):
    edges3 = jnp.stack([edge_index[0].reshape(_E // _CH, _CH),
                        edge_index[1].reshape(_E // _CH, _CH)], axis=1)
    xaug = jnp.concatenate(
        [embeddings, jnp.ones((_N, _DA - _D), jnp.float32)], axis=1)
    z144 = jnp.zeros((_WB, _DA), jnp.float32)
    z128 = jnp.zeros((_WB, _D), jnp.float32)

    agg1 = _sc_aggregate(_DA, xaug, edges3, z144)
    h1, cnt8 = _tc_dense1(agg1, xaug,
                          Wl1, jnp.reshape(bl1, (1, _D)), Wr1)
    agg2 = _sc_aggregate(_D, h1, edges3, z128)
    return _tc_dense2(agg2, cnt8, h1,
                      Wl2, jnp.reshape(bl2, (1, _D)), Wr2)


# speedup vs baseline: 11.6008x; 1.1126x over previous
import jax
import jax.numpy as jnp
from jax import lax
from jax.experimental import pallas as pl
from jax.experimental.pallas import tpu as pltpu
from jax.experimental.pallas import tpu_sc as plsc

_N = 10000
_NP = 10240
_D = 128
_CW = 8
_E = 320000
_NC = 2
_NS = 16
_NW = _NC * _NS
_CH = 128
_NCH = _E // _CH
_NIB = 4
_WB = 80
_RPS = _NP // _NS


def _sc_aggregate(x, edges2, zrow, z8, ones8, with_cnt):
    mesh = plsc.VectorSubcoreMesh(core_axis_name="c", subcore_axis_name="s",
                                  num_cores=_NC, num_subcores=_NS)
    out_type = [jax.ShapeDtypeStruct((_NC, _NP, _D), jnp.float32)]
    scratch = [
        pltpu.VMEM((_NIB, 2, _CH), jnp.int32),
        pltpu.VMEM((2, _CH, _D), jnp.float32),
        pltpu.SemaphoreType.DMA((2,)),
        pltpu.SemaphoreType.DMA((_NIB,)),
        pltpu.SemaphoreType.DMA((2,)),
        pltpu.VMEM_SHARED((_NP, _D), jnp.float32),
    ]
    if with_cnt:
        out_type.append(jax.ShapeDtypeStruct((_NC, _NP, _CW), jnp.float32))
        scratch += [
            pltpu.VMEM((_CH, _CW), jnp.float32),
            pltpu.VMEM((_RPS, _CW), jnp.float32),
            pltpu.VMEM_SHARED((_NP, _CW), jnp.float32),
        ]

    def body(x_hbm, e_hbm, z_hbm, z8_hbm, o_hbm, *refs):
        if with_cnt:
            (agg_hbm, cnt_hbm, idx_v, rows_v, gsem, isem, ssem, agg_sh,
             ones_v, cbuf_v, cnt_sh) = refs
        else:
            agg_hbm, idx_v, rows_v, gsem, isem, ssem, agg_sh = refs
        cid = lax.axis_index("c")
        sid = lax.axis_index("s")
        wid = sid * _NC + cid
        r0 = sid * _RPS

        pltpu.sync_copy(z_hbm, rows_v.at[0, pl.ds(0, _WB)])

        def zstep(j, carry):
            pltpu.async_copy(rows_v.at[0, pl.ds(0, _WB)],
                             agg_sh.at[pl.ds(r0 + j * _WB, _WB)], gsem.at[0])
            return carry

        lax.fori_loop(0, _RPS // _WB, zstep, 0)

        if with_cnt:
            pltpu.sync_copy(o_hbm, ones_v)
            pltpu.sync_copy(z8_hbm, cbuf_v.at[pl.ds(0, _WB)])

            def zcnt(j, carry):
                pltpu.async_copy(cbuf_v.at[pl.ds(0, _WB)],
                                 cnt_sh.at[pl.ds(r0 + j * _WB, _WB)],
                                 ssem.at[0])
                return carry

            lax.fori_loop(0, _RPS // _WB, zcnt, 0)

            def zcnt_drain(j, carry):
                pltpu.make_async_copy(cbuf_v.at[pl.ds(0, _WB)],
                                      cnt_sh.at[pl.ds(r0, _WB)],
                                      ssem.at[0]).wait()
                return carry

            lax.fori_loop(0, _RPS // _WB, zcnt_drain, 0)

        def zdrain(j, carry):
            pltpu.make_async_copy(rows_v.at[0, pl.ds(0, _WB)],
                                  agg_sh.at[pl.ds(r0, _WB)], gsem.at[0]).wait()
            return carry

        lax.fori_loop(0, _RPS // _WB, zdrain, 0)
        plsc.subcore_barrier()

        nch_w = 78 + jnp.where(wid < _NCH - 78 * _NW, 1, 0)

        def start_idx(i, q):
            pltpu.async_copy(e_hbm.at[pl.ds(2 * (wid + _NW * i), 2)],
                             idx_v.at[q], isem.at[q])

        def wait_idx(i, q):
            pltpu.make_async_copy(e_hbm.at[pl.ds(2 * (wid + _NW * i), 2)],
                                  idx_v.at[q], isem.at[q]).wait()

        def start_gather(q, p):
            pltpu.async_copy(x_hbm.at[idx_v.at[q, 0]], rows_v.at[p],
                             gsem.at[p])

        def wait_gather(q, p):
            pltpu.make_async_copy(x_hbm.at[idx_v.at[q, 0]], rows_v.at[p],
                                  gsem.at[p]).wait()

        for i in range(3):
            start_idx(i, i)
        for i in range(2):
            wait_idx(i, i)
            start_gather(i, i)

        def step(i, carry):
            p = lax.rem(i, 2)
            q = lax.rem(i, _NIB)
            wait_gather(q, p)
            pltpu.sync_copy(rows_v.at[p], agg_sh.at[idx_v.at[q, 1]], add=True)
            if with_cnt:
                pltpu.sync_copy(ones_v, cnt_sh.at[idx_v.at[q, 1]], add=True)

            @pl.when(i + 3 < nch_w)
            def _():
                start_idx(i + 3, lax.rem(i + 3, _NIB))

            @pl.when(i + 2 < nch_w)
            def _():
                qq = lax.rem(i + 2, _NIB)
                wait_idx(i + 2, qq)
                start_gather(qq, p)

            return carry

        lax.fori_loop(0, nch_w, step, 0)
        plsc.subcore_barrier()

        def rd(j, p):
            pltpu.async_copy(agg_sh.at[pl.ds(r0 + j * _WB, _WB)],
                             rows_v.at[p, pl.ds(0, _WB)], gsem.at[p])

        def rd_wait(j, p):
            pltpu.make_async_copy(agg_sh.at[pl.ds(r0 + j * _WB, _WB)],
                                  rows_v.at[p, pl.ds(0, _WB)],
                                  gsem.at[p]).wait()

        def wr(j, p):
            pltpu.async_copy(rows_v.at[p, pl.ds(0, _WB)],
                             agg_hbm.at[cid, pl.ds(r0 + j * _WB, _WB)],
                             ssem.at[p])

        def wr_wait(j, p):
            pltpu.make_async_copy(rows_v.at[p, pl.ds(0, _WB)],
                                  agg_hbm.at[cid, pl.ds(r0 + j * _WB, _WB)],
                                  ssem.at[p]).wait()

        rd(0, 0)
        nw = _RPS // _WB

        def wstep(j, carry):
            p = lax.rem(j, 2)
            rd_wait(j, p)
            wr(j, p)

            @pl.when(j + 1 < nw)
            def _():
                p1 = lax.rem(j + 1, 2)

                @pl.when(j >= 1)
                def _():
                    wr_wait(j - 1, p1)

                rd(j + 1, p1)

            return carry

        lax.fori_loop(0, nw, wstep, 0)
        wr_wait(nw - 2, (nw - 2) % 2)
        wr_wait(nw - 1, (nw - 1) % 2)

        if with_cnt:
            pltpu.sync_copy(cnt_sh.at[pl.ds(r0, _RPS)], cbuf_v)
            pltpu.sync_copy(cbuf_v, cnt_hbm.at[cid, pl.ds(r0, _RPS)])

    f = pl.kernel(
        body,
        out_type=tuple(out_type) if with_cnt else out_type[0],
        mesh=mesh,
        compiler_params=pltpu.CompilerParams(use_tc_tiling_on_sc=False),
        scratch_types=tuple(scratch),
    )
    return f(x, edges2, zrow, z8, ones8)


def _tc_dense1(aggp, cntp, x, wl, bl, wr):
    bn = 1000

    def body(a_r, c_r, x_r, wl_r, bl_r, wr_r, o_r, co_r):
        agg = a_r[0] + a_r[1]
        cnt = c_r[0, :, :1] + c_r[1, :, :1]
        mean = agg / jnp.maximum(cnt, 1.0)
        h = (lax.dot_general(mean, wl_r[...], (((1,), (1,)), ((), ())),
                             preferred_element_type=jnp.float32,
                             precision=lax.Precision.HIGHEST)
             + bl_r[...]
             + lax.dot_general(x_r[...], wr_r[...], (((1,), (1,)), ((), ())),
                               preferred_element_type=jnp.float32,
                               precision=lax.Precision.HIGHEST))
        nrm = jnp.sqrt(jnp.sum(h * h, axis=1, keepdims=True))
        o_r[...] = h / jnp.maximum(nrm, 1e-12)
        co_r[...] = jnp.broadcast_to(cnt, (bn, _CW))

    return pl.pallas_call(
        body,
        grid=(_N // bn,),
        in_specs=[
            pl.BlockSpec((2, bn, _D), lambda i: (0, i, 0)),
            pl.BlockSpec((2, bn, _CW), lambda i: (0, i, 0)),
            pl.BlockSpec((bn, _D), lambda i: (i, 0)),
            pl.BlockSpec((_D, _D), lambda i: (0, 0)),
            pl.BlockSpec((1, _D), lambda i: (0, 0)),
            pl.BlockSpec((_D, _D), lambda i: (0, 0)),
        ],
        out_specs=[pl.BlockSpec((bn, _D), lambda i: (i, 0)),
                   pl.BlockSpec((bn, _CW), lambda i: (i, 0))],
        out_shape=[jax.ShapeDtypeStruct((_N, _D), jnp.float32),
                   jax.ShapeDtypeStruct((_N, _CW), jnp.float32)],
    )(aggp, cntp, x, wl, bl, wr)


def _tc_dense2(aggp, cnt8, x, wl, bl, wr):
    bn = 1000

    def body(a_r, c_r, x_r, wl_r, bl_r, wr_r, o_r):
        agg = a_r[0] + a_r[1]
        cnt = c_r[:, :1]
        mean = agg / jnp.maximum(cnt, 1.0)
        h = (lax.dot_general(mean, wl_r[...], (((1,), (1,)), ((), ())),
                             preferred_element_type=jnp.float32,
                             precision=lax.Precision.HIGHEST)
             + bl_r[...]
             + lax.dot_general(x_r[...], wr_r[...], (((1,), (1,)), ((), ())),
                               preferred_element_type=jnp.float32,
                               precision=lax.Precision.HIGHEST))
        nrm = jnp.sqrt(jnp.sum(h * h, axis=1, keepdims=True))
        o_r[...] = h / jnp.maximum(nrm, 1e-12)

    return pl.pallas_call(
        body,
        grid=(_N // bn,),
        in_specs=[
            pl.BlockSpec((2, bn, _D), lambda i: (0, i, 0)),
            pl.BlockSpec((bn, _CW), lambda i: (i, 0)),
            pl.BlockSpec((bn, _D), lambda i: (i, 0)),
            pl.BlockSpec((_D, _D), lambda i: (0, 0)),
            pl.BlockSpec((1, _D), lambda i: (0, 0)),
            pl.BlockSpec((_D, _D), lambda i: (0, 0)),
        ],
        out_specs=pl.BlockSpec((bn, _D), lambda i: (i, 0)),
        out_shape=jax.ShapeDtypeStruct((_N, _D), jnp.float32),
    )(aggp, cnt8, x, wl, bl, wr)


def kernel(embeddings, edge_index, Wl1, bl1, Wr1, Wl2, bl2, Wr2):
    edges2 = jnp.stack([edge_index[0].reshape(_NCH, _CH),
                        edge_index[1].reshape(_NCH, _CH)],
                       axis=1).reshape(2 * _NCH, _CH)
    zrow = jnp.zeros((_WB, _D), jnp.float32)
    z8 = jnp.zeros((_WB, _CW), jnp.float32)
    ones8 = jnp.ones((_CH, _CW), jnp.float32)

    agg1, cnt1 = _sc_aggregate(embeddings, edges2, zrow, z8, ones8,
                               with_cnt=True)
    h1, cnt8 = _tc_dense1(agg1, cnt1, embeddings,
                          Wl1, jnp.reshape(bl1, (1, _D)), Wr1)
    agg2 = _sc_aggregate(h1, edges2, zrow, z8, ones8, with_cnt=False)
    return _tc_dense2(agg2, cnt8, h1,
                      Wl2, jnp.reshape(bl2, (1, _D)), Wr2)


# speedup vs baseline: 11.9306x; 1.0284x over previous
import jax
import jax.numpy as jnp
from jax import lax
from jax.experimental import pallas as pl
from jax.experimental.pallas import tpu as pltpu
from jax.experimental.pallas import tpu_sc as plsc

_N = 10000
_NP = 10240
_D = 128
_CW = 8
_E = 320000
_NC = 2
_NS = 16
_NW = _NC * _NS
_CH = 128
_NCH = _E // _CH
_NIB = 4
_WB = 80
_RPS = _NP // _NS


def _sc_aggregate(x, src, dst, zrow, z8, ones8, with_cnt):
    mesh = plsc.VectorSubcoreMesh(core_axis_name="c", subcore_axis_name="s",
                                  num_cores=_NC, num_subcores=_NS)
    out_type = [jax.ShapeDtypeStruct((_NC, _NP, _D), jnp.float32)]
    scratch = [
        pltpu.VMEM((_NIB, 2, _CH), jnp.int32),
        pltpu.VMEM((2, _CH, _D), jnp.float32),
        pltpu.SemaphoreType.DMA((2,)),
        pltpu.SemaphoreType.DMA((_NIB,)),
        pltpu.SemaphoreType.DMA((2,)),
        pltpu.VMEM_SHARED((_NP, _D), jnp.float32),
    ]
    if with_cnt:
        out_type.append(jax.ShapeDtypeStruct((_NC, _NP, _CW), jnp.float32))
        scratch += [
            pltpu.VMEM((_CH, _CW), jnp.float32),
            pltpu.VMEM((_RPS, _CW), jnp.float32),
            pltpu.VMEM_SHARED((_NP, _CW), jnp.float32),
        ]

    def body(x_hbm, s_hbm, d_hbm, z_hbm, z8_hbm, o_hbm, *refs):
        if with_cnt:
            (agg_hbm, cnt_hbm, idx_v, rows_v, gsem, isem, ssem, agg_sh,
             ones_v, cbuf_v, cnt_sh) = refs
        else:
            agg_hbm, idx_v, rows_v, gsem, isem, ssem, agg_sh = refs
        cid = lax.axis_index("c")
        sid = lax.axis_index("s")
        wid = sid * _NC + cid
        r0 = sid * _RPS

        pltpu.sync_copy(z_hbm, rows_v.at[0, pl.ds(0, _WB)])

        def zstep(j, carry):
            pltpu.async_copy(rows_v.at[0, pl.ds(0, _WB)],
                             agg_sh.at[pl.ds(r0 + j * _WB, _WB)], ssem.at[0])
            return carry

        lax.fori_loop(0, _RPS // _WB, zstep, 0)

        if with_cnt:
            pltpu.sync_copy(o_hbm, ones_v)
            pltpu.sync_copy(z8_hbm, cbuf_v.at[pl.ds(0, _WB)])

            def zcnt(j, carry):
                pltpu.async_copy(cbuf_v.at[pl.ds(0, _WB)],
                                 cnt_sh.at[pl.ds(r0 + j * _WB, _WB)],
                                 ssem.at[1])
                return carry

            lax.fori_loop(0, _RPS // _WB, zcnt, 0)

        nch_w = 78 + jnp.where(wid < _NCH - 78 * _NW, 1, 0)

        def start_idx(i, q):
            e0 = (wid + _NW * i) * _CH
            pltpu.async_copy(s_hbm.at[pl.ds(e0, _CH)], idx_v.at[q, 0],
                             isem.at[q])
            pltpu.async_copy(d_hbm.at[pl.ds(e0, _CH)], idx_v.at[q, 1],
                             isem.at[q])

        def wait_idx(i, q):
            e0 = (wid + _NW * i) * _CH
            pltpu.make_async_copy(s_hbm.at[pl.ds(e0, _CH)], idx_v.at[q, 0],
                                  isem.at[q]).wait()
            pltpu.make_async_copy(d_hbm.at[pl.ds(e0, _CH)], idx_v.at[q, 1],
                                  isem.at[q]).wait()

        def start_gather(q, p):
            pltpu.async_copy(x_hbm.at[idx_v.at[q, 0]], rows_v.at[p],
                             gsem.at[p])

        def wait_gather(q, p):
            pltpu.make_async_copy(x_hbm.at[idx_v.at[q, 0]], rows_v.at[p],
                                  gsem.at[p]).wait()

        for i in range(3):
            start_idx(i, i)

        def zdrain(j, carry):
            pltpu.make_async_copy(rows_v.at[0, pl.ds(0, _WB)],
                                  agg_sh.at[pl.ds(r0, _WB)], ssem.at[0]).wait()
            return carry

        lax.fori_loop(0, _RPS // _WB, zdrain, 0)
        if with_cnt:
            def zcnt_drain(j, carry):
                pltpu.make_async_copy(cbuf_v.at[pl.ds(0, _WB)],
                                      cnt_sh.at[pl.ds(r0, _WB)],
                                      ssem.at[1]).wait()
                return carry

            lax.fori_loop(0, _RPS // _WB, zcnt_drain, 0)
        plsc.subcore_barrier()

        for i in range(2):
            wait_idx(i, i)
            start_gather(i, i)

        def step(i, carry):
            p = lax.rem(i, 2)
            q = lax.rem(i, _NIB)
            wait_gather(q, p)
            pltpu.sync_copy(rows_v.at[p], agg_sh.at[idx_v.at[q, 1]], add=True)
            if with_cnt:
                pltpu.sync_copy(ones_v, cnt_sh.at[idx_v.at[q, 1]], add=True)

            @pl.when(i + 3 < nch_w)
            def _():
                start_idx(i + 3, lax.rem(i + 3, _NIB))

            @pl.when(i + 2 < nch_w)
            def _():
                qq = lax.rem(i + 2, _NIB)
                wait_idx(i + 2, qq)
                start_gather(qq, p)

            return carry

        lax.fori_loop(0, nch_w, step, 0)
        plsc.subcore_barrier()

        def rd(j, p):
            pltpu.async_copy(agg_sh.at[pl.ds(r0 + j * _WB, _WB)],
                             rows_v.at[p, pl.ds(0, _WB)], gsem.at[p])

        def rd_wait(j, p):
            pltpu.make_async_copy(agg_sh.at[pl.ds(r0 + j * _WB, _WB)],
                                  rows_v.at[p, pl.ds(0, _WB)],
                                  gsem.at[p]).wait()

        def wr(j, p):
            pltpu.async_copy(rows_v.at[p, pl.ds(0, _WB)],
                             agg_hbm.at[cid, pl.ds(r0 + j * _WB, _WB)],
                             ssem.at[p])

        def wr_wait(j, p):
            pltpu.make_async_copy(rows_v.at[p, pl.ds(0, _WB)],
                                  agg_hbm.at[cid, pl.ds(r0 + j * _WB, _WB)],
                                  ssem.at[p]).wait()

        rd(0, 0)
        nw = _RPS // _WB

        def wstep(j, carry):
            p = lax.rem(j, 2)
            rd_wait(j, p)
            wr(j, p)

            @pl.when(j + 1 < nw)
            def _():
                p1 = lax.rem(j + 1, 2)

                @pl.when(j >= 1)
                def _():
                    wr_wait(j - 1, p1)

                rd(j + 1, p1)

            return carry

        lax.fori_loop(0, nw, wstep, 0)
        wr_wait(nw - 2, (nw - 2) % 2)
        wr_wait(nw - 1, (nw - 1) % 2)

        if with_cnt:
            pltpu.sync_copy(cnt_sh.at[pl.ds(r0, _RPS)], cbuf_v)
            pltpu.sync_copy(cbuf_v, cnt_hbm.at[cid, pl.ds(r0, _RPS)])

    f = pl.kernel(
        body,
        out_type=tuple(out_type) if with_cnt else out_type[0],
        mesh=mesh,
        compiler_params=pltpu.CompilerParams(use_tc_tiling_on_sc=False),
        scratch_types=tuple(scratch),
    )
    return f(x, src, dst, zrow, z8, ones8)


def _tc_dense1(aggp, cntp, x, wl, bl, wr):
    bn = 1000

    def body(a_r, c_r, x_r, wl_r, bl_r, wr_r, o_r, co_r):
        agg = a_r[0] + a_r[1]
        cnt = c_r[0, :, :1] + c_r[1, :, :1]
        mean = agg / jnp.maximum(cnt, 1.0)
        h = (lax.dot_general(mean, wl_r[...], (((1,), (1,)), ((), ())),
                             preferred_element_type=jnp.float32,
                             precision=lax.Precision.HIGHEST)
             + bl_r[...]
             + lax.dot_general(x_r[...], wr_r[...], (((1,), (1,)), ((), ())),
                               preferred_element_type=jnp.float32,
                               precision=lax.Precision.HIGHEST))
        nrm = jnp.sqrt(jnp.sum(h * h, axis=1, keepdims=True))
        o_r[...] = h / jnp.maximum(nrm, 1e-12)
        co_r[...] = jnp.broadcast_to(cnt, (bn, _CW))

    return pl.pallas_call(
        body,
        grid=(_N // bn,),
        in_specs=[
            pl.BlockSpec((2, bn, _D), lambda i: (0, i, 0)),
            pl.BlockSpec((2, bn, _CW), lambda i: (0, i, 0)),
            pl.BlockSpec((bn, _D), lambda i: (i, 0)),
            pl.BlockSpec((_D, _D), lambda i: (0, 0)),
            pl.BlockSpec((1, _D), lambda i: (0, 0)),
            pl.BlockSpec((_D, _D), lambda i: (0, 0)),
        ],
        out_specs=[pl.BlockSpec((bn, _D), lambda i: (i, 0)),
                   pl.BlockSpec((bn, _CW), lambda i: (i, 0))],
        out_shape=[jax.ShapeDtypeStruct((_N, _D), jnp.float32),
                   jax.ShapeDtypeStruct((_N, _CW), jnp.float32)],
    )(aggp, cntp, x, wl, bl, wr)


def _tc_dense2(aggp, cnt8, x, wl, bl, wr):
    bn = 1000

    def body(a_r, c_r, x_r, wl_r, bl_r, wr_r, o_r):
        agg = a_r[0] + a_r[1]
        cnt = c_r[:, :1]
        mean = agg / jnp.maximum(cnt, 1.0)
        h = (lax.dot_general(mean, wl_r[...], (((1,), (1,)), ((), ())),
                             preferred_element_type=jnp.float32,
                             precision=lax.Precision.HIGHEST)
             + bl_r[...]
             + lax.dot_general(x_r[...], wr_r[...], (((1,), (1,)), ((), ())),
                               preferred_element_type=jnp.float32,
                               precision=lax.Precision.HIGHEST))
        nrm = jnp.sqrt(jnp.sum(h * h, axis=1, keepdims=True))
        o_r[...] = h / jnp.maximum(nrm, 1e-12)

    return pl.pallas_call(
        body,
        grid=(_N // bn,),
        in_specs=[
            pl.BlockSpec((2, bn, _D), lambda i: (0, i, 0)),
            pl.BlockSpec((bn, _CW), lambda i: (i, 0)),
            pl.BlockSpec((bn, _D), lambda i: (i, 0)),
            pl.BlockSpec((_D, _D), lambda i: (0, 0)),
            pl.BlockSpec((1, _D), lambda i: (0, 0)),
            pl.BlockSpec((_D, _D), lambda i: (0, 0)),
        ],
        out_specs=pl.BlockSpec((bn, _D), lambda i: (i, 0)),
        out_shape=jax.ShapeDtypeStruct((_N, _D), jnp.float32),
    )(aggp, cnt8, x, wl, bl, wr)


def kernel(embeddings, edge_index, Wl1, bl1, Wr1, Wl2, bl2, Wr2):
    src = edge_index[0]
    dst = edge_index[1]
    zrow = jnp.zeros((_WB, _D), jnp.float32)
    z8 = jnp.zeros((_WB, _CW), jnp.float32)
    ones8 = jnp.ones((_CH, _CW), jnp.float32)

    agg1, cnt1 = _sc_aggregate(embeddings, src, dst, zrow, z8, ones8,
                               with_cnt=True)
    h1, cnt8 = _tc_dense1(agg1, cnt1, embeddings,
                          Wl1, jnp.reshape(bl1, (1, _D)), Wr1)
    agg2 = _sc_aggregate(h1, src, dst, zrow, z8, ones8, with_cnt=False)
    return _tc_dense2(agg2, cnt8, h1,
                      Wl2, jnp.reshape(bl2, (1, _D)), Wr2)


# speedup vs baseline: 12.3954x; 1.0390x over previous
import jax
import jax.numpy as jnp
from jax import lax
from jax.experimental import pallas as pl
from jax.experimental.pallas import tpu as pltpu
from jax.experimental.pallas import tpu_sc as plsc

_N = 10000
_NP = 10240
_D = 128
_CW = 8
_E = 320000
_NC = 2
_NS = 16
_NW = _NC * _NS
_CH = 80
_NCH = _E // _CH
_CPW = _NCH // _NW
_NIB = 4
_WB = 80
_RPS = _NP // _NS


def _sc_aggregate(x, src, dst, zrow, z8, ones8, with_cnt):
    mesh = plsc.VectorSubcoreMesh(core_axis_name="c", subcore_axis_name="s",
                                  num_cores=_NC, num_subcores=_NS)
    out_type = [jax.ShapeDtypeStruct((_NC, _NP, _D), jnp.float32)]
    scratch = [
        pltpu.VMEM((_NIB, 2, _CH), jnp.int32),
        pltpu.VMEM((3, _CH, _D), jnp.float32),
        pltpu.SemaphoreType.DMA((3,)),
        pltpu.SemaphoreType.DMA((_NIB,)),
        pltpu.SemaphoreType.DMA((3,)),
        pltpu.VMEM_SHARED((_NP, _D), jnp.float32),
    ]
    if with_cnt:
        out_type.append(jax.ShapeDtypeStruct((_NC, _NP, _CW), jnp.float32))
        scratch += [
            pltpu.VMEM((_CH, _CW), jnp.float32),
            pltpu.VMEM((_RPS, _CW), jnp.float32),
            pltpu.VMEM_SHARED((_NP, _CW), jnp.float32),
        ]

    def body(x_hbm, s_hbm, d_hbm, z_hbm, z8_hbm, o_hbm, *refs):
        if with_cnt:
            (agg_hbm, cnt_hbm, idx_v, rows_v, gsem, isem, ssem, agg_sh,
             ones_v, cbuf_v, cnt_sh) = refs
        else:
            agg_hbm, idx_v, rows_v, gsem, isem, ssem, agg_sh = refs
        cid = lax.axis_index("c")
        sid = lax.axis_index("s")
        wid = sid * _NC + cid
        r0 = sid * _RPS

        pltpu.sync_copy(z_hbm, rows_v.at[0, pl.ds(0, _WB)])

        def zstep(j, carry):
            pltpu.async_copy(rows_v.at[0, pl.ds(0, _WB)],
                             agg_sh.at[pl.ds(r0 + j * _WB, _WB)], ssem.at[0])
            return carry

        lax.fori_loop(0, _RPS // _WB, zstep, 0)

        if with_cnt:
            pltpu.sync_copy(o_hbm, ones_v)
            pltpu.sync_copy(z8_hbm, cbuf_v.at[pl.ds(0, _WB)])

            def zcnt(j, carry):
                pltpu.async_copy(cbuf_v.at[pl.ds(0, _WB)],
                                 cnt_sh.at[pl.ds(r0 + j * _WB, _WB)],
                                 ssem.at[1])
                return carry

            lax.fori_loop(0, _RPS // _WB, zcnt, 0)

        def start_idx(i, q):
            e0 = (wid * _CPW + i) * _CH
            pltpu.async_copy(s_hbm.at[pl.ds(e0, _CH)], idx_v.at[q, 0],
                             isem.at[q])
            pltpu.async_copy(d_hbm.at[pl.ds(e0, _CH)], idx_v.at[q, 1],
                             isem.at[q])

        def wait_idx(i, q):
            e0 = (wid * _CPW + i) * _CH
            pltpu.make_async_copy(s_hbm.at[pl.ds(e0, _CH)], idx_v.at[q, 0],
                                  isem.at[q]).wait()
            pltpu.make_async_copy(d_hbm.at[pl.ds(e0, _CH)], idx_v.at[q, 1],
                                  isem.at[q]).wait()

        def start_gather(q, p):
            pltpu.async_copy(x_hbm.at[idx_v.at[q, 0]], rows_v.at[p],
                             gsem.at[p])

        def wait_gather(q, p):
            pltpu.make_async_copy(x_hbm.at[idx_v.at[q, 0]], rows_v.at[p],
                                  gsem.at[p]).wait()

        for i in range(3):
            start_idx(i, i)

        def zdrain(j, carry):
            pltpu.make_async_copy(rows_v.at[0, pl.ds(0, _WB)],
                                  agg_sh.at[pl.ds(r0, _WB)], ssem.at[0]).wait()
            return carry

        lax.fori_loop(0, _RPS // _WB, zdrain, 0)
        if with_cnt:
            def zcnt_drain(j, carry):
                pltpu.make_async_copy(cbuf_v.at[pl.ds(0, _WB)],
                                      cnt_sh.at[pl.ds(r0, _WB)],
                                      ssem.at[1]).wait()
                return carry

            lax.fori_loop(0, _RPS // _WB, zcnt_drain, 0)
        plsc.subcore_barrier()

        for i in range(2):
            wait_idx(i, i)
            start_gather(i, i)

        def start_scatter(q, p):
            pltpu.async_copy(rows_v.at[p], agg_sh.at[idx_v.at[q, 1]],
                             ssem.at[p], add=True)

        def wait_scatter(q, p):
            pltpu.make_async_copy(rows_v.at[p], agg_sh.at[idx_v.at[q, 1]],
                                  ssem.at[p]).wait()

        def step(i, carry):
            p = lax.rem(i, 3)
            q = lax.rem(i, _NIB)
            wait_gather(q, p)
            start_scatter(q, p)
            if with_cnt:
                pltpu.sync_copy(ones_v, cnt_sh.at[idx_v.at[q, 1]], add=True)

            @pl.when(i >= 1)
            def _():
                wait_scatter(lax.rem(i - 1, _NIB), lax.rem(i + 2, 3))

            @pl.when(i + 3 < _CPW)
            def _():
                start_idx(i + 3, lax.rem(i + 3, _NIB))

            @pl.when(i + 2 < _CPW)
            def _():
                qq = lax.rem(i + 2, _NIB)
                wait_idx(i + 2, qq)
                start_gather(qq, lax.rem(i + 2, 3))

            return carry

        lax.fori_loop(0, _CPW, step, 0)
        wait_scatter((_CPW - 1) % _NIB, (_CPW - 1) % 3)
        plsc.subcore_barrier()

        def rd(j, p):
            pltpu.async_copy(agg_sh.at[pl.ds(r0 + j * _WB, _WB)],
                             rows_v.at[p, pl.ds(0, _WB)], gsem.at[p])

        def rd_wait(j, p):
            pltpu.make_async_copy(agg_sh.at[pl.ds(r0 + j * _WB, _WB)],
                                  rows_v.at[p, pl.ds(0, _WB)],
                                  gsem.at[p]).wait()

        def wr(j, p):
            pltpu.async_copy(rows_v.at[p, pl.ds(0, _WB)],
                             agg_hbm.at[cid, pl.ds(r0 + j * _WB, _WB)],
                             ssem.at[p])

        def wr_wait(j, p):
            pltpu.make_async_copy(rows_v.at[p, pl.ds(0, _WB)],
                                  agg_hbm.at[cid, pl.ds(r0 + j * _WB, _WB)],
                                  ssem.at[p]).wait()

        rd(0, 0)
        nw = _RPS // _WB

        def wstep(j, carry):
            p = lax.rem(j, 2)
            rd_wait(j, p)
            wr(j, p)

            @pl.when(j + 1 < nw)
            def _():
                p1 = lax.rem(j + 1, 2)

                @pl.when(j >= 1)
                def _():
                    wr_wait(j - 1, p1)

                rd(j + 1, p1)

            return carry

        lax.fori_loop(0, nw, wstep, 0)
        wr_wait(nw - 2, (nw - 2) % 2)
        wr_wait(nw - 1, (nw - 1) % 2)

        if with_cnt:
            pltpu.sync_copy(cnt_sh.at[pl.ds(r0, _RPS)], cbuf_v)
            pltpu.sync_copy(cbuf_v, cnt_hbm.at[cid, pl.ds(r0, _RPS)])

    f = pl.kernel(
        body,
        out_type=tuple(out_type) if with_cnt else out_type[0],
        mesh=mesh,
        compiler_params=pltpu.CompilerParams(use_tc_tiling_on_sc=False),
        scratch_types=tuple(scratch),
    )
    return f(x, src, dst, zrow, z8, ones8)


def _tc_dense1(aggp, cntp, x, wl, bl, wr):
    bn = 1000

    def body(a_r, c_r, x_r, wl_r, bl_r, wr_r, o_r, co_r):
        agg = a_r[0] + a_r[1]
        cnt = c_r[0, :, :1] + c_r[1, :, :1]
        mean = agg / jnp.maximum(cnt, 1.0)
        h = (lax.dot_general(mean, wl_r[...], (((1,), (1,)), ((), ())),
                             preferred_element_type=jnp.float32,
                             precision=lax.Precision.HIGHEST)
             + bl_r[...]
             + lax.dot_general(x_r[...], wr_r[...], (((1,), (1,)), ((), ())),
                               preferred_element_type=jnp.float32,
                               precision=lax.Precision.HIGHEST))
        nrm = jnp.sqrt(jnp.sum(h * h, axis=1, keepdims=True))
        o_r[...] = h / jnp.maximum(nrm, 1e-12)
        co_r[...] = jnp.broadcast_to(cnt, (bn, _CW))

    return pl.pallas_call(
        body,
        grid=(_N // bn,),
        in_specs=[
            pl.BlockSpec((2, bn, _D), lambda i: (0, i, 0)),
            pl.BlockSpec((2, bn, _CW), lambda i: (0, i, 0)),
            pl.BlockSpec((bn, _D), lambda i: (i, 0)),
            pl.BlockSpec((_D, _D), lambda i: (0, 0)),
            pl.BlockSpec((1, _D), lambda i: (0, 0)),
            pl.BlockSpec((_D, _D), lambda i: (0, 0)),
        ],
        out_specs=[pl.BlockSpec((bn, _D), lambda i: (i, 0)),
                   pl.BlockSpec((bn, _CW), lambda i: (i, 0))],
        out_shape=[jax.ShapeDtypeStruct((_N, _D), jnp.float32),
                   jax.ShapeDtypeStruct((_N, _CW), jnp.float32)],
    )(aggp, cntp, x, wl, bl, wr)


def _tc_dense2(aggp, cnt8, x, wl, bl, wr):
    bn = 1000

    def body(a_r, c_r, x_r, wl_r, bl_r, wr_r, o_r):
        agg = a_r[0] + a_r[1]
        cnt = c_r[:, :1]
        mean = agg / jnp.maximum(cnt, 1.0)
        h = (lax.dot_general(mean, wl_r[...], (((1,), (1,)), ((), ())),
                             preferred_element_type=jnp.float32,
                             precision=lax.Precision.HIGHEST)
             + bl_r[...]
             + lax.dot_general(x_r[...], wr_r[...], (((1,), (1,)), ((), ())),
                               preferred_element_type=jnp.float32,
                               precision=lax.Precision.HIGHEST))
        nrm = jnp.sqrt(jnp.sum(h * h, axis=1, keepdims=True))
        o_r[...] = h / jnp.maximum(nrm, 1e-12)

    return pl.pallas_call(
        body,
        grid=(_N // bn,),
        in_specs=[
            pl.BlockSpec((2, bn, _D), lambda i: (0, i, 0)),
            pl.BlockSpec((bn, _CW), lambda i: (i, 0)),
            pl.BlockSpec((bn, _D), lambda i: (i, 0)),
            pl.BlockSpec((_D, _D), lambda i: (0, 0)),
            pl.BlockSpec((1, _D), lambda i: (0, 0)),
            pl.BlockSpec((_D, _D), lambda i: (0, 0)),
        ],
        out_specs=pl.BlockSpec((bn, _D), lambda i: (i, 0)),
        out_shape=jax.ShapeDtypeStruct((_N, _D), jnp.float32),
    )(aggp, cnt8, x, wl, bl, wr)


def kernel(embeddings, edge_index, Wl1, bl1, Wr1, Wl2, bl2, Wr2):
    src = edge_index[0]
    dst = edge_index[1]
    zrow = jnp.zeros((_WB, _D), jnp.float32)
    z8 = jnp.zeros((_WB, _CW), jnp.float32)
    ones8 = jnp.ones((_CH, _CW), jnp.float32)

    agg1, cnt1 = _sc_aggregate(embeddings, src, dst, zrow, z8, ones8,
                               with_cnt=True)
    h1, cnt8 = _tc_dense1(agg1, cnt1, embeddings,
                          Wl1, jnp.reshape(bl1, (1, _D)), Wr1)
    agg2 = _sc_aggregate(h1, src, dst, zrow, z8, ones8, with_cnt=False)
    return _tc_dense2(agg2, cnt8, h1,
                      Wl2, jnp.reshape(bl2, (1, _D)), Wr2)


# speedup vs baseline: 12.9969x; 1.0485x over previous
import jax
import jax.numpy as jnp
from jax import lax
from jax.experimental import pallas as pl
from jax.experimental.pallas import tpu as pltpu
from jax.experimental.pallas import tpu_sc as plsc

_N = 10000
_NP = 10240
_D = 128
_CW = 8
_E = 320000
_NC = 2
_NS = 16
_NW = _NC * _NS
_CH = 80
_NCH = _E // _CH
_CPW = _NCH // _NW
_NIB = 4
_WB = 80
_RPS = _NP // _NS


def _sc_aggregate(x, src, dst, zrow, z8, ones8, with_cnt):
    mesh = plsc.VectorSubcoreMesh(core_axis_name="c", subcore_axis_name="s",
                                  num_cores=_NC, num_subcores=_NS)
    out_type = [jax.ShapeDtypeStruct((_NC, _NP, _D), jnp.float32)]
    scratch = [
        pltpu.VMEM((_NIB, 2, _CH), jnp.int32),
        pltpu.VMEM((3, _CH, _D), jnp.float32),
        pltpu.SemaphoreType.DMA((3,)),
        pltpu.SemaphoreType.DMA((_NIB,)),
        pltpu.SemaphoreType.DMA((3,)),
        pltpu.VMEM_SHARED((_NP, _D), jnp.float32),
    ]
    if with_cnt:
        out_type.append(jax.ShapeDtypeStruct((_NC, _NP, _CW), jnp.float32))
        scratch += [
            pltpu.VMEM((_CH, _CW), jnp.float32),
            pltpu.VMEM((_RPS, _CW), jnp.float32),
            pltpu.VMEM_SHARED((_NP, _CW), jnp.float32),
        ]

    def body(x_hbm, s_hbm, d_hbm, z_hbm, z8_hbm, o_hbm, *refs):
        if with_cnt:
            (agg_hbm, cnt_hbm, idx_v, rows_v, gsem, isem, ssem, agg_sh,
             ones_v, cbuf_v, cnt_sh) = refs
        else:
            agg_hbm, idx_v, rows_v, gsem, isem, ssem, agg_sh = refs
        cid = lax.axis_index("c")
        sid = lax.axis_index("s")
        wid = sid * _NC + cid
        r0 = sid * _RPS

        pltpu.sync_copy(z_hbm, rows_v.at[0, pl.ds(0, _WB)])

        def zstep(j, carry):
            pltpu.async_copy(rows_v.at[0, pl.ds(0, _WB)],
                             agg_sh.at[pl.ds(r0 + j * _WB, _WB)], ssem.at[0])
            return carry

        lax.fori_loop(0, _RPS // _WB, zstep, 0)

        if with_cnt:
            pltpu.sync_copy(o_hbm, ones_v)
            pltpu.sync_copy(z8_hbm, cbuf_v.at[pl.ds(0, _WB)])

            def zcnt(j, carry):
                pltpu.async_copy(cbuf_v.at[pl.ds(0, _WB)],
                                 cnt_sh.at[pl.ds(r0 + j * _WB, _WB)],
                                 ssem.at[1])
                return carry

            lax.fori_loop(0, _RPS // _WB, zcnt, 0)

        def start_idx(i, q):
            e0 = (wid * _CPW + i) * _CH
            pltpu.async_copy(s_hbm.at[pl.ds(e0, _CH)], idx_v.at[q, 0],
                             isem.at[q])
            pltpu.async_copy(d_hbm.at[pl.ds(e0, _CH)], idx_v.at[q, 1],
                             isem.at[q])

        def wait_idx(i, q):
            e0 = (wid * _CPW + i) * _CH
            pltpu.make_async_copy(s_hbm.at[pl.ds(e0, _CH)], idx_v.at[q, 0],
                                  isem.at[q]).wait()
            pltpu.make_async_copy(d_hbm.at[pl.ds(e0, _CH)], idx_v.at[q, 1],
                                  isem.at[q]).wait()

        def start_gather(q, p):
            pltpu.async_copy(x_hbm.at[idx_v.at[q, 0]], rows_v.at[p],
                             gsem.at[p])

        def wait_gather(q, p):
            pltpu.make_async_copy(x_hbm.at[idx_v.at[q, 0]], rows_v.at[p],
                                  gsem.at[p]).wait()

        for i in range(3):
            start_idx(i, i)

        def zdrain(j, carry):
            pltpu.make_async_copy(rows_v.at[0, pl.ds(0, _WB)],
                                  agg_sh.at[pl.ds(r0, _WB)], ssem.at[0]).wait()
            return carry

        lax.fori_loop(0, _RPS // _WB, zdrain, 0)
        if with_cnt:
            def zcnt_drain(j, carry):
                pltpu.make_async_copy(cbuf_v.at[pl.ds(0, _WB)],
                                      cnt_sh.at[pl.ds(r0, _WB)],
                                      ssem.at[1]).wait()
                return carry

            lax.fori_loop(0, _RPS // _WB, zcnt_drain, 0)
        plsc.subcore_barrier()

        for i in range(2):
            wait_idx(i, i)
            start_gather(i, i)

        def start_scatter(q, p):
            pltpu.async_copy(rows_v.at[p], agg_sh.at[idx_v.at[q, 1]],
                             ssem.at[p], add=True)

        def wait_scatter(q, p):
            pltpu.make_async_copy(rows_v.at[p], agg_sh.at[idx_v.at[q, 1]],
                                  ssem.at[p]).wait()

        def step(i, carry):
            p = lax.rem(i, 3)
            q = lax.rem(i, _NIB)
            wait_gather(q, p)
            start_scatter(q, p)
            if with_cnt:
                pltpu.sync_copy(ones_v, cnt_sh.at[idx_v.at[q, 1]], add=True)

            @pl.when(i >= 1)
            def _():
                wait_scatter(lax.rem(i - 1, _NIB), lax.rem(i + 2, 3))

            @pl.when(i + 3 < _CPW)
            def _():
                start_idx(i + 3, lax.rem(i + 3, _NIB))

            @pl.when(i + 2 < _CPW)
            def _():
                qq = lax.rem(i + 2, _NIB)
                wait_idx(i + 2, qq)
                start_gather(qq, lax.rem(i + 2, 3))

            return carry

        lax.fori_loop(0, _CPW, step, 0)
        wait_scatter((_CPW - 1) % _NIB, (_CPW - 1) % 3)
        plsc.subcore_barrier()

        def rd(j, p):
            pltpu.async_copy(agg_sh.at[pl.ds(r0 + j * _WB, _WB)],
                             rows_v.at[p, pl.ds(0, _WB)], gsem.at[p])

        def rd_wait(j, p):
            pltpu.make_async_copy(agg_sh.at[pl.ds(r0 + j * _WB, _WB)],
                                  rows_v.at[p, pl.ds(0, _WB)],
                                  gsem.at[p]).wait()

        def wr(j, p):
            pltpu.async_copy(rows_v.at[p, pl.ds(0, _WB)],
                             agg_hbm.at[cid, pl.ds(r0 + j * _WB, _WB)],
                             ssem.at[p])

        def wr_wait(j, p):
            pltpu.make_async_copy(rows_v.at[p, pl.ds(0, _WB)],
                                  agg_hbm.at[cid, pl.ds(r0 + j * _WB, _WB)],
                                  ssem.at[p]).wait()

        rd(0, 0)
        nw = _RPS // _WB

        def wstep(j, carry):
            p = lax.rem(j, 2)
            rd_wait(j, p)
            wr(j, p)

            @pl.when(j + 1 < nw)
            def _():
                p1 = lax.rem(j + 1, 2)

                @pl.when(j >= 1)
                def _():
                    wr_wait(j - 1, p1)

                rd(j + 1, p1)

            return carry

        lax.fori_loop(0, nw, wstep, 0)
        wr_wait(nw - 2, (nw - 2) % 2)
        wr_wait(nw - 1, (nw - 1) % 2)

        if with_cnt:
            pltpu.sync_copy(cnt_sh.at[pl.ds(r0, _RPS)], cbuf_v)
            pltpu.sync_copy(cbuf_v, cnt_hbm.at[cid, pl.ds(r0, _RPS)])

    f = pl.kernel(
        body,
        out_type=tuple(out_type) if with_cnt else out_type[0],
        mesh=mesh,
        compiler_params=pltpu.CompilerParams(use_tc_tiling_on_sc=False),
        scratch_types=tuple(scratch),
    )
    return f(x, src, dst, zrow, z8, ones8)


def _tc_dense1(aggp, cntp, x, wl, bl, wr):
    bn = 2000

    def body(a_r, c_r, x_r, wl_r, bl_r, wr_r, o_r, co_r):
        agg = a_r[0] + a_r[1]
        cnt = c_r[0, :, :1] + c_r[1, :, :1]
        mean = agg / jnp.maximum(cnt, 1.0)
        h = (lax.dot_general(mean, wl_r[...], (((1,), (1,)), ((), ())),
                             preferred_element_type=jnp.float32,
                             precision=lax.Precision.HIGHEST)
             + bl_r[...]
             + lax.dot_general(x_r[...], wr_r[...], (((1,), (1,)), ((), ())),
                               preferred_element_type=jnp.float32,
                               precision=lax.Precision.HIGHEST))
        nrm = jnp.sqrt(jnp.sum(h * h, axis=1, keepdims=True))
        o_r[...] = h / jnp.maximum(nrm, 1e-12)
        co_r[...] = jnp.broadcast_to(cnt, (bn, _CW))

    return pl.pallas_call(
        body,
        grid=(_N // bn,),
        in_specs=[
            pl.BlockSpec((2, bn, _D), lambda i: (0, i, 0)),
            pl.BlockSpec((2, bn, _CW), lambda i: (0, i, 0)),
            pl.BlockSpec((bn, _D), lambda i: (i, 0)),
            pl.BlockSpec((_D, _D), lambda i: (0, 0)),
            pl.BlockSpec((1, _D), lambda i: (0, 0)),
            pl.BlockSpec((_D, _D), lambda i: (0, 0)),
        ],
        out_specs=[pl.BlockSpec((bn, _D), lambda i: (i, 0)),
                   pl.BlockSpec((bn, _CW), lambda i: (i, 0))],
        out_shape=[jax.ShapeDtypeStruct((_N, _D), jnp.float32),
                   jax.ShapeDtypeStruct((_N, _CW), jnp.float32)],
    )(aggp, cntp, x, wl, bl, wr)


def _tc_dense2(aggp, cnt8, x, wl, bl, wr):
    bn = 2000

    def body(a_r, c_r, x_r, wl_r, bl_r, wr_r, o_r):
        agg = a_r[0] + a_r[1]
        cnt = c_r[:, :1]
        mean = agg / jnp.maximum(cnt, 1.0)
        h = (lax.dot_general(mean, wl_r[...], (((1,), (1,)), ((), ())),
                             preferred_element_type=jnp.float32,
                             precision=lax.Precision.HIGHEST)
             + bl_r[...]
             + lax.dot_general(x_r[...], wr_r[...], (((1,), (1,)), ((), ())),
                               preferred_element_type=jnp.float32,
                               precision=lax.Precision.HIGHEST))
        nrm = jnp.sqrt(jnp.sum(h * h, axis=1, keepdims=True))
        o_r[...] = h / jnp.maximum(nrm, 1e-12)

    return pl.pallas_call(
        body,
        grid=(_N // bn,),
        in_specs=[
            pl.BlockSpec((2, bn, _D), lambda i: (0, i, 0)),
            pl.BlockSpec((bn, _CW), lambda i: (i, 0)),
            pl.BlockSpec((bn, _D), lambda i: (i, 0)),
            pl.BlockSpec((_D, _D), lambda i: (0, 0)),
            pl.BlockSpec((1, _D), lambda i: (0, 0)),
            pl.BlockSpec((_D, _D), lambda i: (0, 0)),
        ],
        out_specs=pl.BlockSpec((bn, _D), lambda i: (i, 0)),
        out_shape=jax.ShapeDtypeStruct((_N, _D), jnp.float32),
    )(aggp, cnt8, x, wl, bl, wr)


def kernel(embeddings, edge_index, Wl1, bl1, Wr1, Wl2, bl2, Wr2):
    src = edge_index[0]
    dst = edge_index[1]
    zrow = jnp.zeros((_WB, _D), jnp.float32)
    z8 = jnp.zeros((_WB, _CW), jnp.float32)
    ones8 = jnp.ones((_CH, _CW), jnp.float32)

    agg1, cnt1 = _sc_aggregate(embeddings, src, dst, zrow, z8, ones8,
                               with_cnt=True)
    h1, cnt8 = _tc_dense1(agg1, cnt1, embeddings,
                          Wl1, jnp.reshape(bl1, (1, _D)), Wr1)
    agg2 = _sc_aggregate(h1, src, dst, zrow, z8, ones8, with_cnt=False)
    return _tc_dense2(agg2, cnt8, h1,
                      Wl2, jnp.reshape(bl2, (1, _D)), Wr2)


# speedup vs baseline: 13.0495x; 1.0040x over previous
import jax
import jax.numpy as jnp
from jax import lax
from jax.experimental import pallas as pl
from jax.experimental.pallas import tpu as pltpu
from jax.experimental.pallas import tpu_sc as plsc

_N = 10000
_NP = 10240
_D = 128
_CW = 8
_E = 320000
_NC = 2
_NS = 16
_NW = _NC * _NS
_CH = 80
_NCH = _E // _CH
_CPW = _NCH // _NW
_NIB = 4
_WB = 80
_RPS = _NP // _NS


def _sc_aggregate(x, src, dst, zrow, z8, ones8, with_cnt):
    mesh = plsc.VectorSubcoreMesh(core_axis_name="c", subcore_axis_name="s",
                                  num_cores=_NC, num_subcores=_NS)
    out_type = [jax.ShapeDtypeStruct((_NC, _NP, _D), jnp.float32)]
    scratch = [
        pltpu.VMEM((_NIB, 2, _CH), jnp.int32),
        pltpu.VMEM((3, _CH, _D), jnp.float32),
        pltpu.SemaphoreType.DMA((3,)),
        pltpu.SemaphoreType.DMA((_NIB,)),
        pltpu.SemaphoreType.DMA((3,)),
        pltpu.VMEM_SHARED((_NP, _D), jnp.float32),
    ]
    if with_cnt:
        out_type.append(jax.ShapeDtypeStruct((_NC, _NP, _CW), jnp.float32))
        scratch += [
            pltpu.VMEM((_CH, _CW), jnp.float32),
            pltpu.VMEM((_RPS, _CW), jnp.float32),
            pltpu.VMEM_SHARED((_NP, _CW), jnp.float32),
        ]

    def body(x_hbm, s_hbm, d_hbm, z_hbm, z8_hbm, o_hbm, *refs):
        if with_cnt:
            (agg_hbm, cnt_hbm, idx_v, rows_v, gsem, isem, ssem, agg_sh,
             ones_v, cbuf_v, cnt_sh) = refs
        else:
            agg_hbm, idx_v, rows_v, gsem, isem, ssem, agg_sh = refs
        cid = lax.axis_index("c")
        sid = lax.axis_index("s")
        wid = sid * _NC + cid
        r0 = sid * _RPS

        pltpu.sync_copy(z_hbm, rows_v.at[0, pl.ds(0, _WB)])

        def zstep(j, carry):
            pltpu.async_copy(rows_v.at[0, pl.ds(0, _WB)],
                             agg_sh.at[pl.ds(r0 + j * _WB, _WB)], ssem.at[0])
            return carry

        lax.fori_loop(0, _RPS // _WB, zstep, 0)

        if with_cnt:
            pltpu.sync_copy(o_hbm, ones_v)
            pltpu.sync_copy(z8_hbm, cbuf_v.at[pl.ds(0, _WB)])

            def zcnt(j, carry):
                pltpu.async_copy(cbuf_v.at[pl.ds(0, _WB)],
                                 cnt_sh.at[pl.ds(r0 + j * _WB, _WB)],
                                 ssem.at[1])
                return carry

            lax.fori_loop(0, _RPS // _WB, zcnt, 0)

        def start_idx(i, q):
            e0 = (wid * _CPW + i) * _CH
            pltpu.async_copy(s_hbm.at[pl.ds(e0, _CH)], idx_v.at[q, 0],
                             isem.at[q])
            pltpu.async_copy(d_hbm.at[pl.ds(e0, _CH)], idx_v.at[q, 1],
                             isem.at[q])

        def wait_idx(i, q):
            e0 = (wid * _CPW + i) * _CH
            pltpu.make_async_copy(s_hbm.at[pl.ds(e0, _CH)], idx_v.at[q, 0],
                                  isem.at[q]).wait()
            pltpu.make_async_copy(d_hbm.at[pl.ds(e0, _CH)], idx_v.at[q, 1],
                                  isem.at[q]).wait()

        def start_gather(q, p):
            pltpu.async_copy(x_hbm.at[idx_v.at[q, 0]], rows_v.at[p],
                             gsem.at[p])

        def wait_gather(q, p):
            pltpu.make_async_copy(x_hbm.at[idx_v.at[q, 0]], rows_v.at[p],
                                  gsem.at[p]).wait()

        for i in range(3):
            start_idx(i, i)

        def zdrain(j, carry):
            pltpu.make_async_copy(rows_v.at[0, pl.ds(0, _WB)],
                                  agg_sh.at[pl.ds(r0, _WB)], ssem.at[0]).wait()
            return carry

        lax.fori_loop(0, _RPS // _WB, zdrain, 0)
        if with_cnt:
            def zcnt_drain(j, carry):
                pltpu.make_async_copy(cbuf_v.at[pl.ds(0, _WB)],
                                      cnt_sh.at[pl.ds(r0, _WB)],
                                      ssem.at[1]).wait()
                return carry

            lax.fori_loop(0, _RPS // _WB, zcnt_drain, 0)
        plsc.subcore_barrier()

        for i in range(2):
            wait_idx(i, i)
            start_gather(i, i)

        def start_scatter(q, p):
            pltpu.async_copy(rows_v.at[p], agg_sh.at[idx_v.at[q, 1]],
                             ssem.at[p], add=True)

        def wait_scatter(q, p):
            pltpu.make_async_copy(rows_v.at[p], agg_sh.at[idx_v.at[q, 1]],
                                  ssem.at[p]).wait()

        def step(i, carry):
            p = lax.rem(i, 3)
            q = lax.rem(i, _NIB)
            wait_gather(q, p)
            start_scatter(q, p)
            if with_cnt:
                pltpu.sync_copy(ones_v, cnt_sh.at[idx_v.at[q, 1]], add=True)

            @pl.when(i >= 1)
            def _():
                wait_scatter(lax.rem(i - 1, _NIB), lax.rem(i + 2, 3))

            @pl.when(i + 3 < _CPW)
            def _():
                start_idx(i + 3, lax.rem(i + 3, _NIB))

            @pl.when(i + 2 < _CPW)
            def _():
                qq = lax.rem(i + 2, _NIB)
                wait_idx(i + 2, qq)
                start_gather(qq, lax.rem(i + 2, 3))

            return carry

        lax.fori_loop(0, _CPW, step, 0)
        wait_scatter((_CPW - 1) % _NIB, (_CPW - 1) % 3)
        plsc.subcore_barrier()

        def rd(j, p):
            pltpu.async_copy(agg_sh.at[pl.ds(r0 + j * _WB, _WB)],
                             rows_v.at[p, pl.ds(0, _WB)], gsem.at[p])

        def rd_wait(j, p):
            pltpu.make_async_copy(agg_sh.at[pl.ds(r0 + j * _WB, _WB)],
                                  rows_v.at[p, pl.ds(0, _WB)],
                                  gsem.at[p]).wait()

        def wr(j, p):
            pltpu.async_copy(rows_v.at[p, pl.ds(0, _WB)],
                             agg_hbm.at[cid, pl.ds(r0 + j * _WB, _WB)],
                             ssem.at[p])

        def wr_wait(j, p):
            pltpu.make_async_copy(rows_v.at[p, pl.ds(0, _WB)],
                                  agg_hbm.at[cid, pl.ds(r0 + j * _WB, _WB)],
                                  ssem.at[p]).wait()

        rd(0, 0)
        nw = _RPS // _WB

        def wstep(j, carry):
            p = lax.rem(j, 2)
            rd_wait(j, p)
            wr(j, p)

            @pl.when(j + 1 < nw)
            def _():
                p1 = lax.rem(j + 1, 2)

                @pl.when(j >= 1)
                def _():
                    wr_wait(j - 1, p1)

                rd(j + 1, p1)

            return carry

        lax.fori_loop(0, nw, wstep, 0)
        wr_wait(nw - 2, (nw - 2) % 2)
        wr_wait(nw - 1, (nw - 1) % 2)

        if with_cnt:
            pltpu.sync_copy(cnt_sh.at[pl.ds(r0, _RPS)], cbuf_v)
            pltpu.sync_copy(cbuf_v, cnt_hbm.at[cid, pl.ds(r0, _RPS)])

    f = pl.kernel(
        body,
        out_type=tuple(out_type) if with_cnt else out_type[0],
        mesh=mesh,
        compiler_params=pltpu.CompilerParams(use_tc_tiling_on_sc=False),
        scratch_types=tuple(scratch),
    )
    return f(x, src, dst, zrow, z8, ones8)


def _tc_right(x, wr, bl):
    bn = 2000

    def body(x_r, wr_r, bl_r, o_r):
        o_r[...] = lax.dot_general(
            x_r[...], wr_r[...], (((1,), (1,)), ((), ())),
            preferred_element_type=jnp.float32,
            precision=lax.Precision.HIGHEST) + bl_r[...]

    return pl.pallas_call(
        body,
        grid=(_N // bn,),
        in_specs=[
            pl.BlockSpec((bn, _D), lambda i: (i, 0)),
            pl.BlockSpec((_D, _D), lambda i: (0, 0)),
            pl.BlockSpec((1, _D), lambda i: (0, 0)),
        ],
        out_specs=pl.BlockSpec((bn, _D), lambda i: (i, 0)),
        out_shape=jax.ShapeDtypeStruct((_N, _D), jnp.float32),
    )(x, wr, bl)


def _tc_dense1(aggp, cntp, xr, wl):
    bn = 2000

    def body(a_r, c_r, xr_r, wl_r, o_r, co_r):
        agg = a_r[0] + a_r[1]
        cnt = c_r[0, :, :1] + c_r[1, :, :1]
        mean = agg / jnp.maximum(cnt, 1.0)
        h = lax.dot_general(mean, wl_r[...], (((1,), (1,)), ((), ())),
                            preferred_element_type=jnp.float32,
                            precision=lax.Precision.HIGHEST) + xr_r[...]
        nrm = jnp.sqrt(jnp.sum(h * h, axis=1, keepdims=True))
        o_r[...] = h / jnp.maximum(nrm, 1e-12)
        co_r[...] = jnp.broadcast_to(cnt, (bn, _CW))

    return pl.pallas_call(
        body,
        grid=(_N // bn,),
        in_specs=[
            pl.BlockSpec((2, bn, _D), lambda i: (0, i, 0)),
            pl.BlockSpec((2, bn, _CW), lambda i: (0, i, 0)),
            pl.BlockSpec((bn, _D), lambda i: (i, 0)),
            pl.BlockSpec((_D, _D), lambda i: (0, 0)),
        ],
        out_specs=[pl.BlockSpec((bn, _D), lambda i: (i, 0)),
                   pl.BlockSpec((bn, _CW), lambda i: (i, 0))],
        out_shape=[jax.ShapeDtypeStruct((_N, _D), jnp.float32),
                   jax.ShapeDtypeStruct((_N, _CW), jnp.float32)],
    )(aggp, cntp, xr, wl)


def _tc_dense2(aggp, cnt8, xr, wl):
    bn = 2000

    def body(a_r, c_r, xr_r, wl_r, o_r):
        agg = a_r[0] + a_r[1]
        cnt = c_r[:, :1]
        mean = agg / jnp.maximum(cnt, 1.0)
        h = lax.dot_general(mean, wl_r[...], (((1,), (1,)), ((), ())),
                            preferred_element_type=jnp.float32,
                            precision=lax.Precision.HIGHEST) + xr_r[...]
        nrm = jnp.sqrt(jnp.sum(h * h, axis=1, keepdims=True))
        o_r[...] = h / jnp.maximum(nrm, 1e-12)

    return pl.pallas_call(
        body,
        grid=(_N // bn,),
        in_specs=[
            pl.BlockSpec((2, bn, _D), lambda i: (0, i, 0)),
            pl.BlockSpec((bn, _CW), lambda i: (i, 0)),
            pl.BlockSpec((bn, _D), lambda i: (i, 0)),
            pl.BlockSpec((_D, _D), lambda i: (0, 0)),
        ],
        out_specs=pl.BlockSpec((bn, _D), lambda i: (i, 0)),
        out_shape=jax.ShapeDtypeStruct((_N, _D), jnp.float32),
    )(aggp, cnt8, xr, wl)


def kernel(embeddings, edge_index, Wl1, bl1, Wr1, Wl2, bl2, Wr2):
    src = edge_index[0]
    dst = edge_index[1]
    zrow = jnp.zeros((_WB, _D), jnp.float32)
    z8 = jnp.zeros((_WB, _CW), jnp.float32)
    ones8 = jnp.ones((_CH, _CW), jnp.float32)

    xr1 = _tc_right(embeddings, Wr1, jnp.reshape(bl1, (1, _D)))
    agg1, cnt1 = _sc_aggregate(embeddings, src, dst, zrow, z8, ones8,
                               with_cnt=True)
    h1, cnt8 = _tc_dense1(agg1, cnt1, xr1, Wl1)
    xr2 = _tc_right(h1, Wr2, jnp.reshape(bl2, (1, _D)))
    agg2 = _sc_aggregate(h1, src, dst, zrow, z8, ones8, with_cnt=False)
    return _tc_dense2(agg2, cnt8, xr2, Wl2)
